# baseline (speedup 1.0000x reference)
import functools

import jax
import jax.numpy as jnp
from jax import lax
from jax.experimental import pallas as pl
from jax.experimental.pallas import tpu as pltpu
from jax.experimental.pallas import tpu_sc as plsc

K = 8
EPS = 1e-5

NUM_SC_CORES = 2
NUM_SC_SUBCORES = 16
NUM_WORKERS = NUM_SC_CORES * NUM_SC_SUBCORES
GATHER_CHUNK = 128


KNN_CW = 512


def _knn_body(n, npad, r, xb_ref, xt_ref, idx_ref, d_scr):
    cw = KNN_CW
    nc = npad // cw
    big_d = jnp.float32(3.4e38)
    big_i = jnp.int32(2**30)
    xb = xb_ref[...]
    x2r = jnp.sum(xb * xb, axis=1, keepdims=True)
    iota_c = lax.broadcasted_iota(jnp.int32, (r, cw), 1)

    def body_a(c, carry):
        xtc = xt_ref[:, pl.ds(c * cw, cw)]
        dot = jnp.dot(xb, xtc, preferred_element_type=jnp.float32)
        x2c = jnp.sum(xtc * xtc, axis=0, keepdims=True)
        giota = iota_c + c * cw
        dch = x2r - 2.0 * dot + x2c
        dch = jnp.where(giota >= n, big_d, dch)
        d_scr[:, pl.ds(c * cw, cw)] = dch
        return carry

    lax.fori_loop(0, nc, body_a, 0)

    nv = npad // 128
    lane_iota = lax.broadcasted_iota(jnp.int32, (8, 128), 1)

    def body_sg(sg, carry):
        ms0 = (jnp.full((8, 128), big_d, jnp.float32),) * K
        is0 = (jnp.full((8, 128), big_i, jnp.int32),) * K

        def body_col(cv, st):
            ms, js = st
            v = d_scr[pl.ds(sg * 8, 8), pl.ds(cv * 128, 128)]
            ii = lane_iota + cv * 128
            cs = [v < ms[k] for k in range(K)]
            nm, ni = [], []
            for k in range(K):
                if k == 0:
                    sv, si = v, ii
                else:
                    sv = jnp.where(cs[k - 1], ms[k - 1], v)
                    si = jnp.where(cs[k - 1], js[k - 1], ii)
                nm.append(jnp.where(cs[k], sv, ms[k]))
                ni.append(jnp.where(cs[k], si, js[k]))
            return tuple(nm), tuple(ni)

        ms, js = lax.fori_loop(0, nv, body_col, (ms0, is0))

        cols = []
        for _ in range(K):
            mm = functools.reduce(jnp.minimum, ms)
            rowm = jnp.min(mm, axis=1, keepdims=True)
            cand = functools.reduce(
                jnp.minimum,
                [jnp.where(ms[k] <= rowm, js[k], big_i) for k in range(K)])
            j = jnp.min(cand, axis=1, keepdims=True)
            cols.append(j)
            ms = tuple(
                jnp.where(js[k] == j, big_d, ms[k]) for k in range(K))
        idx_ref[pl.ds(sg * 8, 8), :] = jnp.concatenate(cols, axis=1)
        return carry

    lax.fori_loop(0, r // 8, body_sg, 0)


def _knn_topk(x, row_block):
    n, d = x.shape
    npad = -(-n // KNN_CW) * KNN_CW
    xt = jnp.concatenate(
        [x.T, jnp.zeros((d, npad - n), jnp.float32)], axis=1)
    grid = n // row_block
    return pl.pallas_call(
        functools.partial(_knn_body, n, npad, row_block),
        grid=(grid,),
        in_specs=[
            pl.BlockSpec((row_block, d), lambda i: (i, 0)),
            pl.BlockSpec((d, npad), lambda i: (0, 0)),
        ],
        out_specs=pl.BlockSpec((row_block, K), lambda i: (i, 0)),
        out_shape=jax.ShapeDtypeStruct((n, K), jnp.int32),
        scratch_shapes=[pltpu.VMEM((row_block, npad), jnp.float32)],
    )(x, xt)


def _sc_gather(table, idx_padded, ep):
    n, dh = table.shape
    ew = ep // NUM_WORKERS
    nch = ew // GATHER_CHUNK
    mesh = plsc.VectorSubcoreMesh(
        core_axis_name="c",
        subcore_axis_name="s",
        num_cores=NUM_SC_CORES,
        num_subcores=NUM_SC_SUBCORES,
    )

    @functools.partial(
        pl.kernel,
        mesh=mesh,
        out_type=jax.ShapeDtypeStruct((ep, dh), jnp.float32),
        scratch_types=[
            pltpu.VMEM((GATHER_CHUNK,), jnp.int32),
            pltpu.VMEM((GATHER_CHUNK, dh), jnp.float32),
            pltpu.SemaphoreType.DMA,
        ],
    )
    def gather(table_hbm, idx_hbm, out_hbm, idx_v, rows_v, sem):
        wid = lax.axis_index("s") * NUM_SC_CORES + lax.axis_index("c")
        base = wid * ew

        def body(c, carry):
            start = base + c * GATHER_CHUNK
            pltpu.sync_copy(idx_hbm.at[pl.ds(start, GATHER_CHUNK)], idx_v)
            pltpu.async_copy(table_hbm.at[idx_v], rows_v, sem).wait()
            pltpu.sync_copy(rows_v, out_hbm.at[pl.ds(start, GATHER_CHUNK)])
            return carry

        lax.fori_loop(0, nch, body, 0)

    return gather(table, idx_padded)


def _mlp1_body(din, dh, gpad, x_ref, xj_ref, w_ref, b_ref,
               out_ref, stats_ref, acc):
    xi = x_ref[:, :din]
    w = w_ref[...]
    b = b_ref[...]
    s = jnp.zeros((1, dh), jnp.float32)
    ss = jnp.zeros((1, dh), jnp.float32)
    for r in range(K):
        xj = xj_ref[:, r * gpad:r * gpad + din]
        feat = jnp.concatenate([xi, xj - xi], axis=1)
        q = jnp.dot(feat, w, preferred_element_type=jnp.float32) + b
        out_ref[:, r * dh:(r + 1) * dh] = q
        s = s + jnp.sum(q, axis=0, keepdims=True)
        ss = ss + jnp.sum(q * q, axis=0, keepdims=True)
    cur = jnp.concatenate([s, ss, jnp.zeros((6, dh), jnp.float32)], axis=0)

    @pl.when(pl.program_id(0) == 0)
    def _():
        acc[...] = cur

    @pl.when(pl.program_id(0) > 0)
    def _():
        acc[...] = acc[...] + cur

    @pl.when(pl.program_id(0) == pl.num_programs(0) - 1)
    def _():
        stats_ref[...] = acc[...]


def _mlp1(xpad, xjv, w1, b1, din, gpad, row_block):
    n = xpad.shape[0]
    dh = w1.shape[1]
    grid = n // row_block
    return pl.pallas_call(
        functools.partial(_mlp1_body, din, dh, gpad),
        grid=(grid,),
        in_specs=[
            pl.BlockSpec((row_block, xpad.shape[1]), lambda i: (i, 0)),
            pl.BlockSpec((row_block, K * gpad), lambda i: (i, 0)),
            pl.BlockSpec((2 * din, dh), lambda i: (0, 0)),
            pl.BlockSpec((1, dh), lambda i: (0, 0)),
        ],
        out_specs=[
            pl.BlockSpec((row_block, K * dh), lambda i: (i, 0)),
            pl.BlockSpec((8, dh), lambda i: (0, 0)),
        ],
        out_shape=[
            jax.ShapeDtypeStruct((n, K * dh), jnp.float32),
            jax.ShapeDtypeStruct((8, dh), jnp.float32),
        ],
        scratch_shapes=[pltpu.VMEM((8, dh), jnp.float32)],
    )(xpad, xjv, w1, b1.reshape(1, dh))


def _bn_coeffs(stats, count):
    mean = stats[0:1, :] / count
    var = stats[1:2, :] / count - mean * mean
    sq = jnp.sqrt(var + EPS)
    return mean, sq


def _bn_relu(h, mean, sq, g, be):
    return jnp.maximum(g * (h - mean) / sq + be, 0.0)


def _layer_body(dh, cw, count, *refs):
    (in_ref, stats_ref, g_ref, be_ref, w_ref, b_ref,
     out_ref, ostats_ref, acc) = refs
    mean, sq = _bn_coeffs(stats_ref[...], count)
    g = g_ref[...]
    be = be_ref[...]
    w = w_ref[...]
    b = b_ref[...]
    s = jnp.zeros((1, dh), jnp.float32)
    ss = jnp.zeros((1, dh), jnp.float32)
    for r in range(K):
        p = in_ref[:, r * cw:r * cw + dh]
        h = _bn_relu(p, mean, sq, g, be)
        q = jnp.dot(h, w, preferred_element_type=jnp.float32) + b
        out_ref[:, r * dh:(r + 1) * dh] = q
        s = s + jnp.sum(q, axis=0, keepdims=True)
        ss = ss + jnp.sum(q * q, axis=0, keepdims=True)
    cur = jnp.concatenate([s, ss, jnp.zeros((6, dh), jnp.float32)], axis=0)

    @pl.when(pl.program_id(0) == 0)
    def _():
        acc[...] = cur

    @pl.when(pl.program_id(0) > 0)
    def _():
        acc[...] = acc[...] + cur

    @pl.when(pl.program_id(0) == pl.num_programs(0) - 1)
    def _():
        ostats_ref[...] = acc[...]


def _layer(inv, stats, g, be, w, b, count, cw, row_block):
    n = inv.shape[0]
    dh = w.shape[0]
    grid = n // row_block
    in_specs = [pl.BlockSpec((row_block, K * cw), lambda i: (i, 0))]
    args = [inv]
    in_specs += [
        pl.BlockSpec((8, dh), lambda i: (0, 0)),
        pl.BlockSpec((1, dh), lambda i: (0, 0)),
        pl.BlockSpec((1, dh), lambda i: (0, 0)),
        pl.BlockSpec((dh, dh), lambda i: (0, 0)),
        pl.BlockSpec((1, dh), lambda i: (0, 0)),
    ]
    args += [stats, g.reshape(1, dh), be.reshape(1, dh), w, b.reshape(1, dh)]
    return pl.pallas_call(
        functools.partial(_layer_body, dh, cw, count),
        grid=(grid,),
        in_specs=in_specs,
        out_specs=[
            pl.BlockSpec((row_block, K * dh), lambda i: (i, 0)),
            pl.BlockSpec((8, dh), lambda i: (0, 0)),
        ],
        out_shape=[
            jax.ShapeDtypeStruct((n, K * dh), jnp.float32),
            jax.ShapeDtypeStruct((8, dh), jnp.float32),
        ],
        scratch_shapes=[pltpu.VMEM((8, dh), jnp.float32)],
    )(*args)


def _agg_body(dh, dout, count, in_ref, stats_ref, g_ref, be_ref, out_ref):
    mean, sq = _bn_coeffs(stats_ref[...], count)
    g = g_ref[...]
    be = be_ref[...]
    acc = jnp.zeros((in_ref.shape[0], dh), jnp.float32)
    for r in range(K):
        p = in_ref[:, r * dh:(r + 1) * dh]
        acc = acc + _bn_relu(p, mean, sq, g, be)
    if dout > dh:
        acc = jnp.concatenate(
            [acc, jnp.zeros((acc.shape[0], dout - dh), jnp.float32)], axis=1)
    out_ref[...] = acc


def _agg(inv, stats, g, be, count, dout, row_block):
    n = inv.shape[0]
    dh = g.shape[0]
    grid = n // row_block
    return pl.pallas_call(
        functools.partial(_agg_body, dh, dout, count),
        grid=(grid,),
        in_specs=[
            pl.BlockSpec((row_block, K * dh), lambda i: (i, 0)),
            pl.BlockSpec((8, dh), lambda i: (0, 0)),
            pl.BlockSpec((1, dh), lambda i: (0, 0)),
            pl.BlockSpec((1, dh), lambda i: (0, 0)),
        ],
        out_specs=pl.BlockSpec((row_block, dout), lambda i: (i, 0)),
        out_shape=jax.ShapeDtypeStruct((n, dout), jnp.float32),
    )(inv, stats, g.reshape(1, dh), be.reshape(1, dh))


def _head_body(h_ref, w1_ref, b1_ref, w2_ref, b2_ref, out_ref):
    t = jnp.dot(h_ref[...], w1_ref[...], preferred_element_type=jnp.float32)
    t = jnp.maximum(t + b1_ref[...], 0.0)
    o = jnp.dot(t, w2_ref[...], preferred_element_type=jnp.float32)
    o = o + b2_ref[...]
    out_ref[...] = 1.0 / (1.0 + jnp.exp(-o))


def _head(h, ew1, eb1, ew2, eb2, row_block):
    n, dh = h.shape
    dmid = ew1.shape[1]
    grid = n // row_block
    return pl.pallas_call(
        _head_body,
        grid=(grid,),
        in_specs=[
            pl.BlockSpec((row_block, dh), lambda i: (i, 0)),
            pl.BlockSpec((dh, dmid), lambda i: (0, 0)),
            pl.BlockSpec((1, dmid), lambda i: (0, 0)),
            pl.BlockSpec((dmid, 1), lambda i: (0, 0)),
            pl.BlockSpec((1, 1), lambda i: (0, 0)),
        ],
        out_specs=pl.BlockSpec((row_block, 1), lambda i: (i, 0)),
        out_shape=jax.ShapeDtypeStruct((n, 1), jnp.float32),
    )(h, ew1, eb1.reshape(1, dmid), ew2, eb2.reshape(1, 1))


def _edge_block(xpad, p, din, dout, knn_block, row_block):
    n, gpad = xpad.shape
    dh = p['w1'].shape[1]
    count = float(n * K)

    idx = _knn_topk(xpad, knn_block)

    e = n * K
    ep = -(-e // (NUM_WORKERS * GATHER_CHUNK)) * (NUM_WORKERS * GATHER_CHUNK)
    idx_flat = idx.reshape(-1)
    idx_padded = jnp.concatenate(
        [idx_flat, jnp.zeros((ep - e,), jnp.int32)])
    xj = _sc_gather(xpad, idx_padded, ep)
    xjv = xj[:e].reshape(n, K * gpad)

    p1v, stats1 = _mlp1(xpad, xjv, p['w1'], p['b1'], din, gpad, row_block)
    h2v, stats2 = _layer(p1v, stats1, p['g1'], p['be1'], p['w2'],
                         p['b2'], count, dh, row_block)
    h3v, stats3 = _layer(h2v, stats2, p['g2'], p['be2'], p['w3'],
                         p['b3'], count, dh, row_block)
    return _agg(h3v, stats3, p['g3'], p['be3'], count, dout, row_block)


def kernel(X, params):
    h = _edge_block(X, params['conv1'], din=128, dout=128,
                    knn_block=400, row_block=1000)
    h = _edge_block(h, params['conv2'], din=64, dout=128,
                    knn_block=400, row_block=1000)
    out = _head(h, params['ew1'], params['eb1'], params['ew2'],
                params['eb2'], row_block=1000)
    return out[:, 0]


# speedup vs baseline: 1.5666x; 1.5666x over previous
import functools

import jax
import jax.numpy as jnp
from jax import lax
from jax.experimental import pallas as pl
from jax.experimental.pallas import tpu as pltpu
from jax.experimental.pallas import tpu_sc as plsc

K = 8
EPS = 1e-5

NUM_SC_CORES = 2
NUM_SC_SUBCORES = 16
NUM_WORKERS = NUM_SC_CORES * NUM_SC_SUBCORES
GATHER_CHUNK = 128


KNN_CW = 512


def _knn_body(n, npad, r, xb_ref, xt_ref, idx_ref, d_scr):
    cw = KNN_CW
    nc = npad // cw
    big_d = jnp.float32(3.4e38)
    big_i = jnp.int32(2**30)
    xb = xb_ref[...]
    x2r = jnp.sum(xb * xb, axis=1, keepdims=True)
    iota_c = lax.broadcasted_iota(jnp.int32, (r, cw), 1)

    def body_a(c, carry):
        xtc = xt_ref[:, pl.ds(c * cw, cw)]
        dot = jnp.dot(xb, xtc, preferred_element_type=jnp.float32)
        x2c = jnp.sum(xtc * xtc, axis=0, keepdims=True)
        giota = iota_c + c * cw
        dch = x2r - 2.0 * dot + x2c
        dch = jnp.where(giota >= n, big_d, dch)
        d_scr[:, pl.ds(c * cw, cw)] = dch
        return carry

    lax.fori_loop(0, nc, body_a, 0)

    nv = npad // 128
    lane_iota = lax.broadcasted_iota(jnp.int32, (8, 128), 1)

    def _insert(st, v, ii):
        ms, js = st
        cs = [v < ms[k] for k in range(K)]
        nm, ni = [], []
        for k in range(K):
            if k == 0:
                sv, si = v, ii
            else:
                sv = jnp.where(cs[k - 1], ms[k - 1], v)
                si = jnp.where(cs[k - 1], js[k - 1], ii)
            nm.append(jnp.where(cs[k], sv, ms[k]))
            ni.append(jnp.where(cs[k], si, js[k]))
        return tuple(nm), tuple(ni)

    def _merge_cols(st):
        ms, js = st
        ms, js = list(ms), list(js)
        cols = []
        for _ in range(K):
            rowm = jnp.min(ms[0], axis=1, keepdims=True)
            cand = jnp.where(ms[0] <= rowm, js[0], big_i)
            j = jnp.min(cand, axis=1, keepdims=True)
            cols.append(j)
            win = js[0] == j
            for k in range(K - 1):
                ms[k] = jnp.where(win, ms[k + 1], ms[k])
                js[k] = jnp.where(win, js[k + 1], js[k])
            ms[K - 1] = jnp.where(win, big_d, ms[K - 1])
        return jnp.concatenate(cols, axis=1)

    ms0 = (jnp.full((8, 128), big_d, jnp.float32),) * K
    is0 = (jnp.full((8, 128), big_i, jnp.int32),) * K

    def body_sg(sgp, carry):
        base = sgp * 16

        def body_col(cv, st):
            sta, stb = st
            ii = lane_iota + cv * 128
            va = d_scr[pl.ds(base, 8), pl.ds(cv * 128, 128)]
            vb = d_scr[pl.ds(base + 8, 8), pl.ds(cv * 128, 128)]
            return _insert(sta, va, ii), _insert(stb, vb, ii)

        sta, stb = lax.fori_loop(0, nv, body_col, ((ms0, is0), (ms0, is0)))
        idx_ref[pl.ds(base, 8), :] = _merge_cols(sta)
        idx_ref[pl.ds(base + 8, 8), :] = _merge_cols(stb)
        return carry

    lax.fori_loop(0, r // 16, body_sg, 0)


def _knn_topk(x, row_block):
    n, d = x.shape
    npad = -(-n // KNN_CW) * KNN_CW
    xt = jnp.concatenate(
        [x.T, jnp.zeros((d, npad - n), jnp.float32)], axis=1)
    grid = n // row_block
    return pl.pallas_call(
        functools.partial(_knn_body, n, npad, row_block),
        grid=(grid,),
        in_specs=[
            pl.BlockSpec((row_block, d), lambda i: (i, 0)),
            pl.BlockSpec((d, npad), lambda i: (0, 0)),
        ],
        out_specs=pl.BlockSpec((row_block, K), lambda i: (i, 0)),
        out_shape=jax.ShapeDtypeStruct((n, K), jnp.int32),
        scratch_shapes=[pltpu.VMEM((row_block, npad), jnp.float32)],
    )(x, xt)


def _sc_gather(table, idx_padded, ep):
    n, dh = table.shape
    ew = ep // NUM_WORKERS
    nch = ew // GATHER_CHUNK
    mesh = plsc.VectorSubcoreMesh(
        core_axis_name="c",
        subcore_axis_name="s",
        num_cores=NUM_SC_CORES,
        num_subcores=NUM_SC_SUBCORES,
    )

    @functools.partial(
        pl.kernel,
        mesh=mesh,
        out_type=jax.ShapeDtypeStruct((ep, dh), jnp.float32),
        scratch_types=[
            pltpu.VMEM((GATHER_CHUNK,), jnp.int32),
            pltpu.VMEM((GATHER_CHUNK, dh), jnp.float32),
            pltpu.SemaphoreType.DMA,
        ],
    )
    def gather(table_hbm, idx_hbm, out_hbm, idx_v, rows_v, sem):
        wid = lax.axis_index("s") * NUM_SC_CORES + lax.axis_index("c")
        base = wid * ew

        def body(c, carry):
            start = base + c * GATHER_CHUNK
            pltpu.sync_copy(idx_hbm.at[pl.ds(start, GATHER_CHUNK)], idx_v)
            pltpu.async_copy(table_hbm.at[idx_v], rows_v, sem).wait()
            pltpu.sync_copy(rows_v, out_hbm.at[pl.ds(start, GATHER_CHUNK)])
            return carry

        lax.fori_loop(0, nch, body, 0)

    return gather(table, idx_padded)


def _mlp1_body(din, dh, gpad, x_ref, xj_ref, w_ref, b_ref,
               out_ref, stats_ref, acc):
    xi = x_ref[:, :din]
    w = w_ref[...]
    b = b_ref[...]
    s = jnp.zeros((1, dh), jnp.float32)
    ss = jnp.zeros((1, dh), jnp.float32)
    for r in range(K):
        xj = xj_ref[:, r * gpad:r * gpad + din]
        feat = jnp.concatenate([xi, xj - xi], axis=1)
        q = jnp.dot(feat, w, preferred_element_type=jnp.float32) + b
        out_ref[:, r * dh:(r + 1) * dh] = q
        s = s + jnp.sum(q, axis=0, keepdims=True)
        ss = ss + jnp.sum(q * q, axis=0, keepdims=True)
    cur = jnp.concatenate([s, ss, jnp.zeros((6, dh), jnp.float32)], axis=0)

    @pl.when(pl.program_id(0) == 0)
    def _():
        acc[...] = cur

    @pl.when(pl.program_id(0) > 0)
    def _():
        acc[...] = acc[...] + cur

    @pl.when(pl.program_id(0) == pl.num_programs(0) - 1)
    def _():
        stats_ref[...] = acc[...]


def _mlp1(xpad, xjv, w1, b1, din, gpad, row_block):
    n = xpad.shape[0]
    dh = w1.shape[1]
    grid = n // row_block
    return pl.pallas_call(
        functools.partial(_mlp1_body, din, dh, gpad),
        grid=(grid,),
        in_specs=[
            pl.BlockSpec((row_block, xpad.shape[1]), lambda i: (i, 0)),
            pl.BlockSpec((row_block, K * gpad), lambda i: (i, 0)),
            pl.BlockSpec((2 * din, dh), lambda i: (0, 0)),
            pl.BlockSpec((1, dh), lambda i: (0, 0)),
        ],
        out_specs=[
            pl.BlockSpec((row_block, K * dh), lambda i: (i, 0)),
            pl.BlockSpec((8, dh), lambda i: (0, 0)),
        ],
        out_shape=[
            jax.ShapeDtypeStruct((n, K * dh), jnp.float32),
            jax.ShapeDtypeStruct((8, dh), jnp.float32),
        ],
        scratch_shapes=[pltpu.VMEM((8, dh), jnp.float32)],
    )(xpad, xjv, w1, b1.reshape(1, dh))


def _bn_coeffs(stats, count):
    mean = stats[0:1, :] / count
    var = stats[1:2, :] / count - mean * mean
    sq = jnp.sqrt(var + EPS)
    return mean, sq


def _bn_relu(h, mean, sq, g, be):
    return jnp.maximum(g * (h - mean) / sq + be, 0.0)


def _layer_body(dh, cw, count, *refs):
    (in_ref, stats_ref, g_ref, be_ref, w_ref, b_ref,
     out_ref, ostats_ref, acc) = refs
    mean, sq = _bn_coeffs(stats_ref[...], count)
    g = g_ref[...]
    be = be_ref[...]
    w = w_ref[...]
    b = b_ref[...]
    s = jnp.zeros((1, dh), jnp.float32)
    ss = jnp.zeros((1, dh), jnp.float32)
    for r in range(K):
        p = in_ref[:, r * cw:r * cw + dh]
        h = _bn_relu(p, mean, sq, g, be)
        q = jnp.dot(h, w, preferred_element_type=jnp.float32) + b
        out_ref[:, r * dh:(r + 1) * dh] = q
        s = s + jnp.sum(q, axis=0, keepdims=True)
        ss = ss + jnp.sum(q * q, axis=0, keepdims=True)
    cur = jnp.concatenate([s, ss, jnp.zeros((6, dh), jnp.float32)], axis=0)

    @pl.when(pl.program_id(0) == 0)
    def _():
        acc[...] = cur

    @pl.when(pl.program_id(0) > 0)
    def _():
        acc[...] = acc[...] + cur

    @pl.when(pl.program_id(0) == pl.num_programs(0) - 1)
    def _():
        ostats_ref[...] = acc[...]


def _layer(inv, stats, g, be, w, b, count, cw, row_block):
    n = inv.shape[0]
    dh = w.shape[0]
    grid = n // row_block
    in_specs = [pl.BlockSpec((row_block, K * cw), lambda i: (i, 0))]
    args = [inv]
    in_specs += [
        pl.BlockSpec((8, dh), lambda i: (0, 0)),
        pl.BlockSpec((1, dh), lambda i: (0, 0)),
        pl.BlockSpec((1, dh), lambda i: (0, 0)),
        pl.BlockSpec((dh, dh), lambda i: (0, 0)),
        pl.BlockSpec((1, dh), lambda i: (0, 0)),
    ]
    args += [stats, g.reshape(1, dh), be.reshape(1, dh), w, b.reshape(1, dh)]
    return pl.pallas_call(
        functools.partial(_layer_body, dh, cw, count),
        grid=(grid,),
        in_specs=in_specs,
        out_specs=[
            pl.BlockSpec((row_block, K * dh), lambda i: (i, 0)),
            pl.BlockSpec((8, dh), lambda i: (0, 0)),
        ],
        out_shape=[
            jax.ShapeDtypeStruct((n, K * dh), jnp.float32),
            jax.ShapeDtypeStruct((8, dh), jnp.float32),
        ],
        scratch_shapes=[pltpu.VMEM((8, dh), jnp.float32)],
    )(*args)


def _agg_body(dh, dout, count, in_ref, stats_ref, g_ref, be_ref, out_ref):
    mean, sq = _bn_coeffs(stats_ref[...], count)
    g = g_ref[...]
    be = be_ref[...]
    acc = jnp.zeros((in_ref.shape[0], dh), jnp.float32)
    for r in range(K):
        p = in_ref[:, r * dh:(r + 1) * dh]
        acc = acc + _bn_relu(p, mean, sq, g, be)
    if dout > dh:
        acc = jnp.concatenate(
            [acc, jnp.zeros((acc.shape[0], dout - dh), jnp.float32)], axis=1)
    out_ref[...] = acc


def _agg(inv, stats, g, be, count, dout, row_block):
    n = inv.shape[0]
    dh = g.shape[0]
    grid = n // row_block
    return pl.pallas_call(
        functools.partial(_agg_body, dh, dout, count),
        grid=(grid,),
        in_specs=[
            pl.BlockSpec((row_block, K * dh), lambda i: (i, 0)),
            pl.BlockSpec((8, dh), lambda i: (0, 0)),
            pl.BlockSpec((1, dh), lambda i: (0, 0)),
            pl.BlockSpec((1, dh), lambda i: (0, 0)),
        ],
        out_specs=pl.BlockSpec((row_block, dout), lambda i: (i, 0)),
        out_shape=jax.ShapeDtypeStruct((n, dout), jnp.float32),
    )(inv, stats, g.reshape(1, dh), be.reshape(1, dh))


def _head_body(h_ref, w1_ref, b1_ref, w2_ref, b2_ref, out_ref):
    t = jnp.dot(h_ref[...], w1_ref[...], preferred_element_type=jnp.float32)
    t = jnp.maximum(t + b1_ref[...], 0.0)
    o = jnp.dot(t, w2_ref[...], preferred_element_type=jnp.float32)
    o = o + b2_ref[...]
    out_ref[...] = 1.0 / (1.0 + jnp.exp(-o))


def _head(h, ew1, eb1, ew2, eb2, row_block):
    n, dh = h.shape
    dmid = ew1.shape[1]
    grid = n // row_block
    return pl.pallas_call(
        _head_body,
        grid=(grid,),
        in_specs=[
            pl.BlockSpec((row_block, dh), lambda i: (i, 0)),
            pl.BlockSpec((dh, dmid), lambda i: (0, 0)),
            pl.BlockSpec((1, dmid), lambda i: (0, 0)),
            pl.BlockSpec((dmid, 1), lambda i: (0, 0)),
            pl.BlockSpec((1, 1), lambda i: (0, 0)),
        ],
        out_specs=pl.BlockSpec((row_block, 1), lambda i: (i, 0)),
        out_shape=jax.ShapeDtypeStruct((n, 1), jnp.float32),
    )(h, ew1, eb1.reshape(1, dmid), ew2, eb2.reshape(1, 1))


def _edge_block(xpad, p, din, dout, knn_block, row_block):
    n, gpad = xpad.shape
    dh = p['w1'].shape[1]
    count = float(n * K)

    idx = _knn_topk(xpad, knn_block)

    e = n * K
    ep = -(-e // (NUM_WORKERS * GATHER_CHUNK)) * (NUM_WORKERS * GATHER_CHUNK)
    idx_flat = idx.reshape(-1)
    idx_padded = jnp.concatenate(
        [idx_flat, jnp.zeros((ep - e,), jnp.int32)])
    xj = _sc_gather(xpad, idx_padded, ep)
    xjv = xj[:e].reshape(n, K * gpad)

    p1v, stats1 = _mlp1(xpad, xjv, p['w1'], p['b1'], din, gpad, row_block)
    h2v, stats2 = _layer(p1v, stats1, p['g1'], p['be1'], p['w2'],
                         p['b2'], count, dh, row_block)
    h3v, stats3 = _layer(h2v, stats2, p['g2'], p['be2'], p['w3'],
                         p['b3'], count, dh, row_block)
    return _agg(h3v, stats3, p['g3'], p['be3'], count, dout, row_block)


def kernel(X, params):
    h = _edge_block(X, params['conv1'], din=128, dout=128,
                    knn_block=400, row_block=1000)
    h = _edge_block(h, params['conv2'], din=64, dout=128,
                    knn_block=400, row_block=1000)
    out = _head(h, params['ew1'], params['eb1'], params['ew2'],
                params['eb2'], row_block=1000)
    return out[:, 0]


# speedup vs baseline: 1.6462x; 1.0508x over previous
import functools

import jax
import jax.numpy as jnp
from jax import lax
from jax.experimental import pallas as pl
from jax.experimental.pallas import tpu as pltpu
from jax.experimental.pallas import tpu_sc as plsc

K = 8
EPS = 1e-5

NUM_SC_CORES = 2
NUM_SC_SUBCORES = 16
NUM_WORKERS = NUM_SC_CORES * NUM_SC_SUBCORES
GATHER_CHUNK = 128


KNN_CW = 512


def _knn_body(n, npad, r, xb_ref, xt_ref, idx_ref, d_scr):
    cw = KNN_CW
    nc = npad // cw
    big_d = jnp.float32(3.4e38)
    big_i = jnp.int32(2**30)
    xb = xb_ref[...]
    x2r = jnp.sum(xb * xb, axis=1, keepdims=True)
    iota_c = lax.broadcasted_iota(jnp.int32, (r, cw), 1)

    def body_a(c, carry):
        xtc = xt_ref[:, pl.ds(c * cw, cw)]
        dot = jnp.dot(xb, xtc, preferred_element_type=jnp.float32)
        x2c = jnp.sum(xtc * xtc, axis=0, keepdims=True)
        giota = iota_c + c * cw
        dch = x2r - 2.0 * dot + x2c
        dch = jnp.where(giota >= n, big_d, dch)
        d_scr[:, pl.ds(c * cw, cw)] = dch
        return carry

    lax.fori_loop(0, nc, body_a, 0)

    nv = npad // 128
    lane_iota = lax.broadcasted_iota(jnp.int32, (8, 128), 1)

    def _insert(st, v, ii):
        ms, js = st
        cs = [v < ms[k] for k in range(K)]
        nm, ni = [], []
        for k in range(K):
            if k == 0:
                sv, si = v, ii
            else:
                sv = jnp.where(cs[k - 1], ms[k - 1], v)
                si = jnp.where(cs[k - 1], js[k - 1], ii)
            nm.append(jnp.where(cs[k], sv, ms[k]))
            ni.append(jnp.where(cs[k], si, js[k]))
        return tuple(nm), tuple(ni)

    def _merge_cols(st):
        ms, js = st
        ms, js = list(ms), list(js)
        cols = []
        for _ in range(K):
            rowm = jnp.min(ms[0], axis=1, keepdims=True)
            cand = jnp.where(ms[0] <= rowm, js[0], big_i)
            j = jnp.min(cand, axis=1, keepdims=True)
            cols.append(j)
            win = js[0] == j
            for k in range(K - 1):
                ms[k] = jnp.where(win, ms[k + 1], ms[k])
                js[k] = jnp.where(win, js[k + 1], js[k])
            ms[K - 1] = jnp.where(win, big_d, ms[K - 1])
        return jnp.concatenate(cols, axis=1)

    ms0 = (jnp.full((8, 128), big_d, jnp.float32),) * K
    is0 = (jnp.full((8, 128), big_i, jnp.int32),) * K

    def body_sg(sgp, carry):
        base = sgp * 16

        def body_col(cv, st):
            sta, stb, va, vb = st
            ii = lane_iota + cv * 128
            nxt = jnp.minimum(cv + 1, nv - 1) * 128
            nva = d_scr[pl.ds(base, 8), pl.ds(nxt, 128)]
            nvb = d_scr[pl.ds(base + 8, 8), pl.ds(nxt, 128)]
            return _insert(sta, va, ii), _insert(stb, vb, ii), nva, nvb

        va0 = d_scr[pl.ds(base, 8), pl.ds(0, 128)]
        vb0 = d_scr[pl.ds(base + 8, 8), pl.ds(0, 128)]
        sta, stb, _, _ = lax.fori_loop(
            0, nv, body_col, ((ms0, is0), (ms0, is0), va0, vb0))
        idx_ref[pl.ds(base, 8), :] = _merge_cols(sta)
        idx_ref[pl.ds(base + 8, 8), :] = _merge_cols(stb)
        return carry

    lax.fori_loop(0, r // 16, body_sg, 0)


def _knn_topk(x, row_block):
    n, d = x.shape
    npad = -(-n // KNN_CW) * KNN_CW
    xt = jnp.concatenate(
        [x.T, jnp.zeros((d, npad - n), jnp.float32)], axis=1)
    grid = n // row_block
    return pl.pallas_call(
        functools.partial(_knn_body, n, npad, row_block),
        grid=(grid,),
        in_specs=[
            pl.BlockSpec((row_block, d), lambda i: (i, 0)),
            pl.BlockSpec((d, npad), lambda i: (0, 0)),
        ],
        out_specs=pl.BlockSpec((row_block, K), lambda i: (i, 0)),
        out_shape=jax.ShapeDtypeStruct((n, K), jnp.int32),
        scratch_shapes=[pltpu.VMEM((row_block, npad), jnp.float32)],
    )(x, xt)


def _sc_gather(table, idx_padded, ep):
    n, dh = table.shape
    ew = ep // NUM_WORKERS
    nch = ew // GATHER_CHUNK
    mesh = plsc.VectorSubcoreMesh(
        core_axis_name="c",
        subcore_axis_name="s",
        num_cores=NUM_SC_CORES,
        num_subcores=NUM_SC_SUBCORES,
    )

    @functools.partial(
        pl.kernel,
        mesh=mesh,
        out_type=jax.ShapeDtypeStruct((ep, dh), jnp.float32),
        scratch_types=[
            pltpu.VMEM((GATHER_CHUNK,), jnp.int32),
            pltpu.VMEM((GATHER_CHUNK, dh), jnp.float32),
            pltpu.SemaphoreType.DMA,
        ],
    )
    def gather(table_hbm, idx_hbm, out_hbm, idx_v, rows_v, sem):
        wid = lax.axis_index("s") * NUM_SC_CORES + lax.axis_index("c")
        base = wid * ew

        def body(c, carry):
            start = base + c * GATHER_CHUNK
            pltpu.sync_copy(idx_hbm.at[pl.ds(start, GATHER_CHUNK)], idx_v)
            pltpu.async_copy(table_hbm.at[idx_v], rows_v, sem).wait()
            pltpu.sync_copy(rows_v, out_hbm.at[pl.ds(start, GATHER_CHUNK)])
            return carry

        lax.fori_loop(0, nch, body, 0)

    return gather(table, idx_padded)


def _mlp1_body(din, dh, gpad, x_ref, xj_ref, w_ref, b_ref,
               out_ref, stats_ref, acc):
    xi = x_ref[:, :din]
    w = w_ref[...]
    b = b_ref[...]
    s = jnp.zeros((1, dh), jnp.float32)
    ss = jnp.zeros((1, dh), jnp.float32)
    for r in range(K):
        xj = xj_ref[:, r * gpad:r * gpad + din]
        feat = jnp.concatenate([xi, xj - xi], axis=1)
        q = jnp.dot(feat, w, preferred_element_type=jnp.float32) + b
        out_ref[:, r * dh:(r + 1) * dh] = q
        s = s + jnp.sum(q, axis=0, keepdims=True)
        ss = ss + jnp.sum(q * q, axis=0, keepdims=True)
    cur = jnp.concatenate([s, ss, jnp.zeros((6, dh), jnp.float32)], axis=0)

    @pl.when(pl.program_id(0) == 0)
    def _():
        acc[...] = cur

    @pl.when(pl.program_id(0) > 0)
    def _():
        acc[...] = acc[...] + cur

    @pl.when(pl.program_id(0) == pl.num_programs(0) - 1)
    def _():
        stats_ref[...] = acc[...]


def _mlp1(xpad, xjv, w1, b1, din, gpad, row_block):
    n = xpad.shape[0]
    dh = w1.shape[1]
    grid = n // row_block
    return pl.pallas_call(
        functools.partial(_mlp1_body, din, dh, gpad),
        grid=(grid,),
        in_specs=[
            pl.BlockSpec((row_block, xpad.shape[1]), lambda i: (i, 0)),
            pl.BlockSpec((row_block, K * gpad), lambda i: (i, 0)),
            pl.BlockSpec((2 * din, dh), lambda i: (0, 0)),
            pl.BlockSpec((1, dh), lambda i: (0, 0)),
        ],
        out_specs=[
            pl.BlockSpec((row_block, K * dh), lambda i: (i, 0)),
            pl.BlockSpec((8, dh), lambda i: (0, 0)),
        ],
        out_shape=[
            jax.ShapeDtypeStruct((n, K * dh), jnp.float32),
            jax.ShapeDtypeStruct((8, dh), jnp.float32),
        ],
        scratch_shapes=[pltpu.VMEM((8, dh), jnp.float32)],
    )(xpad, xjv, w1, b1.reshape(1, dh))


def _bn_coeffs(stats, count):
    mean = stats[0:1, :] / count
    var = stats[1:2, :] / count - mean * mean
    sq = jnp.sqrt(var + EPS)
    return mean, sq


def _bn_relu(h, mean, sq, g, be):
    return jnp.maximum(g * (h - mean) / sq + be, 0.0)


def _layer_body(dh, cw, count, *refs):
    (in_ref, stats_ref, g_ref, be_ref, w_ref, b_ref,
     out_ref, ostats_ref, acc) = refs
    mean, sq = _bn_coeffs(stats_ref[...], count)
    g = g_ref[...]
    be = be_ref[...]
    w = w_ref[...]
    b = b_ref[...]
    s = jnp.zeros((1, dh), jnp.float32)
    ss = jnp.zeros((1, dh), jnp.float32)
    for r in range(K):
        p = in_ref[:, r * cw:r * cw + dh]
        h = _bn_relu(p, mean, sq, g, be)
        q = jnp.dot(h, w, preferred_element_type=jnp.float32) + b
        out_ref[:, r * dh:(r + 1) * dh] = q
        s = s + jnp.sum(q, axis=0, keepdims=True)
        ss = ss + jnp.sum(q * q, axis=0, keepdims=True)
    cur = jnp.concatenate([s, ss, jnp.zeros((6, dh), jnp.float32)], axis=0)

    @pl.when(pl.program_id(0) == 0)
    def _():
        acc[...] = cur

    @pl.when(pl.program_id(0) > 0)
    def _():
        acc[...] = acc[...] + cur

    @pl.when(pl.program_id(0) == pl.num_programs(0) - 1)
    def _():
        ostats_ref[...] = acc[...]


def _layer(inv, stats, g, be, w, b, count, cw, row_block):
    n = inv.shape[0]
    dh = w.shape[0]
    grid = n // row_block
    in_specs = [pl.BlockSpec((row_block, K * cw), lambda i: (i, 0))]
    args = [inv]
    in_specs += [
        pl.BlockSpec((8, dh), lambda i: (0, 0)),
        pl.BlockSpec((1, dh), lambda i: (0, 0)),
        pl.BlockSpec((1, dh), lambda i: (0, 0)),
        pl.BlockSpec((dh, dh), lambda i: (0, 0)),
        pl.BlockSpec((1, dh), lambda i: (0, 0)),
    ]
    args += [stats, g.reshape(1, dh), be.reshape(1, dh), w, b.reshape(1, dh)]
    return pl.pallas_call(
        functools.partial(_layer_body, dh, cw, count),
        grid=(grid,),
        in_specs=in_specs,
        out_specs=[
            pl.BlockSpec((row_block, K * dh), lambda i: (i, 0)),
            pl.BlockSpec((8, dh), lambda i: (0, 0)),
        ],
        out_shape=[
            jax.ShapeDtypeStruct((n, K * dh), jnp.float32),
            jax.ShapeDtypeStruct((8, dh), jnp.float32),
        ],
        scratch_shapes=[pltpu.VMEM((8, dh), jnp.float32)],
    )(*args)


def _agg_body(dh, dout, count, in_ref, stats_ref, g_ref, be_ref, out_ref):
    mean, sq = _bn_coeffs(stats_ref[...], count)
    g = g_ref[...]
    be = be_ref[...]
    acc = jnp.zeros((in_ref.shape[0], dh), jnp.float32)
    for r in range(K):
        p = in_ref[:, r * dh:(r + 1) * dh]
        acc = acc + _bn_relu(p, mean, sq, g, be)
    if dout > dh:
        acc = jnp.concatenate(
            [acc, jnp.zeros((acc.shape[0], dout - dh), jnp.float32)], axis=1)
    out_ref[...] = acc


def _agg(inv, stats, g, be, count, dout, row_block):
    n = inv.shape[0]
    dh = g.shape[0]
    grid = n // row_block
    return pl.pallas_call(
        functools.partial(_agg_body, dh, dout, count),
        grid=(grid,),
        in_specs=[
            pl.BlockSpec((row_block, K * dh), lambda i: (i, 0)),
            pl.BlockSpec((8, dh), lambda i: (0, 0)),
            pl.BlockSpec((1, dh), lambda i: (0, 0)),
            pl.BlockSpec((1, dh), lambda i: (0, 0)),
        ],
        out_specs=pl.BlockSpec((row_block, dout), lambda i: (i, 0)),
        out_shape=jax.ShapeDtypeStruct((n, dout), jnp.float32),
    )(inv, stats, g.reshape(1, dh), be.reshape(1, dh))


def _head_body(h_ref, w1_ref, b1_ref, w2_ref, b2_ref, out_ref):
    t = jnp.dot(h_ref[...], w1_ref[...], preferred_element_type=jnp.float32)
    t = jnp.maximum(t + b1_ref[...], 0.0)
    o = jnp.dot(t, w2_ref[...], preferred_element_type=jnp.float32)
    o = o + b2_ref[...]
    out_ref[...] = 1.0 / (1.0 + jnp.exp(-o))


def _head(h, ew1, eb1, ew2, eb2, row_block):
    n, dh = h.shape
    dmid = ew1.shape[1]
    grid = n // row_block
    return pl.pallas_call(
        _head_body,
        grid=(grid,),
        in_specs=[
            pl.BlockSpec((row_block, dh), lambda i: (i, 0)),
            pl.BlockSpec((dh, dmid), lambda i: (0, 0)),
            pl.BlockSpec((1, dmid), lambda i: (0, 0)),
            pl.BlockSpec((dmid, 1), lambda i: (0, 0)),
            pl.BlockSpec((1, 1), lambda i: (0, 0)),
        ],
        out_specs=pl.BlockSpec((row_block, 1), lambda i: (i, 0)),
        out_shape=jax.ShapeDtypeStruct((n, 1), jnp.float32),
    )(h, ew1, eb1.reshape(1, dmid), ew2, eb2.reshape(1, 1))


def _edge_block(xpad, p, din, dout, knn_block, row_block):
    n, gpad = xpad.shape
    dh = p['w1'].shape[1]
    count = float(n * K)

    idx = _knn_topk(xpad, knn_block)

    e = n * K
    ep = -(-e // (NUM_WORKERS * GATHER_CHUNK)) * (NUM_WORKERS * GATHER_CHUNK)
    idx_flat = idx.reshape(-1)
    idx_padded = jnp.concatenate(
        [idx_flat, jnp.zeros((ep - e,), jnp.int32)])
    xj = _sc_gather(xpad, idx_padded, ep)
    xjv = xj[:e].reshape(n, K * gpad)

    p1v, stats1 = _mlp1(xpad, xjv, p['w1'], p['b1'], din, gpad, row_block)
    h2v, stats2 = _layer(p1v, stats1, p['g1'], p['be1'], p['w2'],
                         p['b2'], count, dh, row_block)
    h3v, stats3 = _layer(h2v, stats2, p['g2'], p['be2'], p['w3'],
                         p['b3'], count, dh, row_block)
    return _agg(h3v, stats3, p['g3'], p['be3'], count, dout, row_block)


def kernel(X, params):
    h = _edge_block(X, params['conv1'], din=128, dout=128,
                    knn_block=400, row_block=1000)
    h = _edge_block(h, params['conv2'], din=64, dout=128,
                    knn_block=400, row_block=1000)
    out = _head(h, params['ew1'], params['eb1'], params['ew2'],
                params['eb2'], row_block=1000)
    return out[:, 0]
